# baseline (speedup 1.0000x reference)
import functools

import jax
import jax.numpy as jnp
from jax import lax
from jax.experimental import pallas as pl
from jax.experimental.pallas import tpu as pltpu
from jax.experimental.pallas import tpu_sc as plsc

N = 10000
H = 256
S = 8
NC = 2
NS = 16
TPW0 = 448
TPW1 = 192
NPAD = NS * (TPW0 + TPW1)
BASE1 = NS * TPW0
C = 8
NBUF = 4
BLK = 1000
GRID = N // BLK


def _sc_gather_sums(h1, h2, idx0, idx1):
    mesh = plsc.VectorSubcoreMesh(core_axis_name="c", subcore_axis_name="s")

    @functools.partial(
        pl.kernel,
        out_type=(
            jax.ShapeDtypeStruct((NPAD, H), jnp.float32),
            jax.ShapeDtypeStruct((NPAD, H), jnp.float32),
        ),
        mesh=mesh,
        scratch_types=(
            [pltpu.VMEM((TPW0 * S,), jnp.int32)]
            + [pltpu.VMEM((C * S, H), jnp.float32)] * NBUF
            + [pltpu.VMEM((C, H), jnp.float32)] * NBUF
            + [pltpu.SemaphoreType.DMA] * (2 * NBUF)
        ),
    )
    def sc_kernel(h1_hbm, h2_hbm, i0_hbm, i1_hbm, o0_hbm, o1_hbm,
                  idx_v, *bufs):
        rows = bufs[0:NBUF]
        accs = bufs[NBUF:2 * NBUF]
        gsem = bufs[2 * NBUF:3 * NBUF]
        wsem = bufs[3 * NBUF:4 * NBUF]
        core = lax.axis_index("c")
        sid = lax.axis_index("s")

        for ci, tpw in ((0, TPW0), (1, TPW1)):
            chunks = tpw // C

            @pl.when(core == ci)
            def _(ci=ci, tpw=tpw, chunks=chunks):
                tbase = sid * tpw + (BASE1 if ci == 1 else 0)
                ibase = tbase * S

                for t_hbm, i_hbm, o_hbm in ((h1_hbm, i0_hbm, o0_hbm),
                                            (h2_hbm, i1_hbm, o1_hbm)):
                    pltpu.sync_copy(i_hbm.at[pl.ds(ibase, tpw * S)],
                                    idx_v.at[pl.ds(0, tpw * S)])

                    def g_start(cc, b, t_hbm=t_hbm):
                        pltpu.make_async_copy(
                            t_hbm.at[idx_v.at[pl.ds(cc * (C * S), C * S)]],
                            rows[b], gsem[b]).start()

                    def g_wait(b, t_hbm=t_hbm):
                        pltpu.make_async_copy(
                            t_hbm.at[idx_v.at[pl.ds(0, C * S)]],
                            rows[b], gsem[b]).wait()

                    def w_start(cc, b, o_hbm=o_hbm, tbase=tbase):
                        pltpu.make_async_copy(
                            accs[b], o_hbm.at[pl.ds(tbase + cc * C, C)],
                            wsem[b]).start()

                    def w_wait(b, o_hbm=o_hbm, tbase=tbase):
                        pltpu.make_async_copy(
                            accs[b], o_hbm.at[pl.ds(tbase, C)],
                            wsem[b]).wait()

                    for pre in range(NBUF - 1):
                        g_start(pre, pre)

                    @pl.loop(0, chunks, step=NBUF)
                    def _(c, g_start=g_start, g_wait=g_wait,
                          w_start=w_start, w_wait=w_wait, chunks=chunks):
                        for b in range(NBUF):
                            cc = c + b
                            nxt = cc + NBUF - 1

                            @pl.when(nxt < chunks)
                            def _(nxt=nxt, b=b):
                                g_start(nxt, (b + NBUF - 1) % NBUF)

                            g_wait(b)

                            @pl.when(cc >= NBUF)
                            def _(b=b):
                                w_wait(b)

                            rb, ab = rows[b], accs[b]

                            @pl.loop(0, C)
                            def _(t, rb=rb, ab=ab):
                                r = t * S
                                for j in range(H // 16):
                                    sl = pl.ds(j * 16, 16)
                                    v01 = rb[r, sl] + rb[r + 1, sl]
                                    v23 = rb[r + 2, sl] + rb[r + 3, sl]
                                    v45 = rb[r + 4, sl] + rb[r + 5, sl]
                                    v67 = rb[r + 6, sl] + rb[r + 7, sl]
                                    ab[t, sl] = (v01 + v23) + (v45 + v67)

                            w_start(cc, b)

                    for b in range(NBUF):
                        w_wait(b)

    return sc_kernel(h1, h2, idx0, idx1)


def _tc_colsums(s0, s1, fc_wt, fc_b):
    def body(x0_ref, x1_ref, w_ref, b_ref, out_ref):
        @pl.when(pl.program_id(0) == 0)
        def _():
            out_ref[...] = jnp.zeros_like(out_ref)

        t0 = jnp.tanh(jnp.dot(x0_ref[...], w_ref[...],
                              preferred_element_type=jnp.float32) + b_ref[...])
        t1 = jnp.tanh(jnp.dot(x1_ref[...], w_ref[...],
                              preferred_element_type=jnp.float32) + b_ref[...])
        out_ref[0:1, :] += jnp.sum(t0, axis=0, keepdims=True)
        out_ref[1:2, :] += jnp.sum(t1, axis=0, keepdims=True)

    return pl.pallas_call(
        body,
        grid=(GRID,),
        in_specs=[
            pl.BlockSpec((BLK, H), lambda i: (i, 0)),
            pl.BlockSpec((BLK, H), lambda i: (i, 0)),
            pl.BlockSpec((H, H), lambda i: (0, 0)),
            pl.BlockSpec((1, H), lambda i: (0, 0)),
        ],
        out_specs=pl.BlockSpec((8, H), lambda i: (0, 0)),
        out_shape=jax.ShapeDtypeStruct((8, H), jnp.float32),
    )(s0, s1, fc_wt, fc_b)


def _tc_combine(cs, att, s0, s1, pred_wt, pred_b):
    def body(cs_ref, att_ref, x0_ref, x1_ref, w_ref, b_ref, out_ref):
        a = att_ref[0, :]
        v0 = jnp.sum(cs_ref[0, :] * a) * (1.0 / N)
        v1 = jnp.sum(cs_ref[1, :] * a) * (1.0 / N)
        m = jnp.maximum(v0, v1)
        e0 = jnp.exp(v0 - m)
        e1 = jnp.exp(v1 - m)
        inv = 1.0 / (e0 + e1)
        b0 = e0 * inv
        b1 = e1 * inv
        z = x0_ref[...] * b0 + x1_ref[...] * b1
        out_ref[...] = jnp.tanh(
            jnp.dot(z, w_ref[...], preferred_element_type=jnp.float32)
            + b_ref[...])

    return pl.pallas_call(
        body,
        grid=(GRID,),
        in_specs=[
            pl.BlockSpec((8, H), lambda i: (0, 0)),
            pl.BlockSpec((1, H), lambda i: (0, 0)),
            pl.BlockSpec((BLK, H), lambda i: (i, 0)),
            pl.BlockSpec((BLK, H), lambda i: (i, 0)),
            pl.BlockSpec((H, H), lambda i: (0, 0)),
            pl.BlockSpec((1, H), lambda i: (0, 0)),
        ],
        out_specs=pl.BlockSpec((BLK, H), lambda i: (i, 0)),
        out_shape=jax.ShapeDtypeStruct((N, H), jnp.float32),
    )(cs, att, s0, s1, pred_wt, pred_b)


def kernel(h0, h1, h2, nei_idx0, nei_idx1, fc_W, fc_b, att, pred_W, pred_b):
    del h0
    idx0 = nei_idx0.astype(jnp.int32).reshape(-1)
    idx1 = nei_idx1.astype(jnp.int32).reshape(-1)
    pad = NPAD * S - idx0.shape[0]
    idx0 = jnp.concatenate([idx0, jnp.zeros((pad,), jnp.int32)])
    idx1 = jnp.concatenate([idx1, jnp.zeros((pad,), jnp.int32)])

    s0, s1 = _sc_gather_sums(h1, h2, idx0, idx1)

    fc_wt = fc_W.T * (1.0 / S)
    pred_wt = pred_W.T * (1.0 / S)
    cs = _tc_colsums(s0, s1, fc_wt, fc_b.reshape(1, H))
    out = _tc_combine(cs, att.reshape(1, H), s0, s1,
                      pred_wt, pred_b.reshape(1, H))
    return out


# speedup vs baseline: 1.0608x; 1.0608x over previous
import functools

import jax
import jax.numpy as jnp
from jax import lax
from jax.experimental import pallas as pl
from jax.experimental.pallas import tpu as pltpu
from jax.experimental.pallas import tpu_sc as plsc

N = 10000
H = 256
S = 8
NC = 2
NS = 16
TPW0 = 512
TPW1 = 128
NPAD = NS * (TPW0 + TPW1)
BASE1 = NS * TPW0
C = 8
NBUF = 4
BLK = 1000
GRID = N // BLK


def _sc_gather_sums(h1, h2, idx0, idx1):
    mesh = plsc.VectorSubcoreMesh(core_axis_name="c", subcore_axis_name="s")

    @functools.partial(
        pl.kernel,
        out_type=(
            jax.ShapeDtypeStruct((NPAD, H), jnp.float32),
            jax.ShapeDtypeStruct((NPAD, H), jnp.float32),
        ),
        mesh=mesh,
        scratch_types=(
            [pltpu.VMEM((2 * TPW0 * S,), jnp.int32)]
            + [pltpu.VMEM((C * S, H), jnp.float32)] * NBUF
            + [pltpu.VMEM((C, H), jnp.float32)] * NBUF
            + [pltpu.SemaphoreType.DMA] * (2 * NBUF + 2)
        ),
    )
    def sc_kernel(h1_hbm, h2_hbm, i0_hbm, i1_hbm, o0_hbm, o1_hbm,
                  idx_v, *bufs):
        rows = bufs[0:NBUF]
        accs = bufs[NBUF:2 * NBUF]
        gsem = bufs[2 * NBUF:3 * NBUF]
        wsem = bufs[3 * NBUF:4 * NBUF]
        isem = bufs[4 * NBUF:4 * NBUF + 2]
        core = lax.axis_index("c")
        sid = lax.axis_index("s")
        ioff = TPW0 * S

        for ci, tpw in ((0, TPW0), (1, TPW1)):
            chunks = tpw // C

            @pl.when(core == ci)
            def _(ci=ci, tpw=tpw, chunks=chunks):
                tbase = sid * tpw + (BASE1 if ci == 1 else 0)
                ibase = tbase * S

                def i_copy(i_hbm, tab, tab_isem):
                    return pltpu.make_async_copy(
                        i_hbm.at[pl.ds(ibase, tpw * S)],
                        idx_v.at[pl.ds(tab * ioff, tpw * S)], tab_isem)

                i_copy(i0_hbm, 0, isem[0]).start()
                i_copy(i1_hbm, 1, isem[1]).start()

                for tab, (t_hbm, i_hbm, o_hbm) in enumerate(
                        ((h1_hbm, i0_hbm, o0_hbm), (h2_hbm, i1_hbm, o1_hbm))):
                    i_copy(i_hbm, tab, isem[tab]).wait()

                    def g_start(cc, b, t_hbm=t_hbm, tab=tab):
                        pltpu.make_async_copy(
                            t_hbm.at[idx_v.at[
                                pl.ds(tab * ioff + cc * (C * S), C * S)]],
                            rows[b], gsem[b]).start()

                    def g_wait(b, t_hbm=t_hbm):
                        pltpu.make_async_copy(
                            t_hbm.at[idx_v.at[pl.ds(0, C * S)]],
                            rows[b], gsem[b]).wait()

                    def w_start(cc, b, o_hbm=o_hbm, tbase=tbase):
                        pltpu.make_async_copy(
                            accs[b], o_hbm.at[pl.ds(tbase + cc * C, C)],
                            wsem[b]).start()

                    def w_wait(b, o_hbm=o_hbm, tbase=tbase):
                        pltpu.make_async_copy(
                            accs[b], o_hbm.at[pl.ds(tbase, C)],
                            wsem[b]).wait()

                    for pre in range(NBUF - 1):
                        g_start(pre, pre)

                    @pl.loop(0, chunks, step=NBUF)
                    def _(c, g_start=g_start, g_wait=g_wait,
                          w_start=w_start, w_wait=w_wait, chunks=chunks):
                        for b in range(NBUF):
                            cc = c + b
                            nxt = cc + NBUF - 1

                            @pl.when(nxt < chunks)
                            def _(nxt=nxt, b=b):
                                g_start(nxt, (b + NBUF - 1) % NBUF)

                            g_wait(b)

                            @pl.when(cc >= NBUF)
                            def _(b=b):
                                w_wait(b)

                            rb, ab = rows[b], accs[b]

                            @pl.loop(0, C)
                            def _(t, rb=rb, ab=ab):
                                r = t * S
                                for j in range(H // 16):
                                    sl = pl.ds(j * 16, 16)
                                    v01 = rb[r, sl] + rb[r + 1, sl]
                                    v23 = rb[r + 2, sl] + rb[r + 3, sl]
                                    v45 = rb[r + 4, sl] + rb[r + 5, sl]
                                    v67 = rb[r + 6, sl] + rb[r + 7, sl]
                                    ab[t, sl] = (v01 + v23) + (v45 + v67)

                            w_start(cc, b)

                    for b in range(NBUF):
                        w_wait(b)

    return sc_kernel(h1, h2, idx0, idx1)


def _tc_colsums(s0, s1, fc_wt, fc_b):
    def body(x0_ref, x1_ref, w_ref, b_ref, out_ref):
        @pl.when(pl.program_id(0) == 0)
        def _():
            out_ref[...] = jnp.zeros_like(out_ref)

        t0 = jnp.tanh(jnp.dot(x0_ref[...], w_ref[...],
                              preferred_element_type=jnp.float32) + b_ref[...])
        t1 = jnp.tanh(jnp.dot(x1_ref[...], w_ref[...],
                              preferred_element_type=jnp.float32) + b_ref[...])
        out_ref[0:1, :] += jnp.sum(t0, axis=0, keepdims=True)
        out_ref[1:2, :] += jnp.sum(t1, axis=0, keepdims=True)

    return pl.pallas_call(
        body,
        grid=(GRID,),
        in_specs=[
            pl.BlockSpec((BLK, H), lambda i: (i, 0)),
            pl.BlockSpec((BLK, H), lambda i: (i, 0)),
            pl.BlockSpec((H, H), lambda i: (0, 0)),
            pl.BlockSpec((1, H), lambda i: (0, 0)),
        ],
        out_specs=pl.BlockSpec((8, H), lambda i: (0, 0)),
        out_shape=jax.ShapeDtypeStruct((8, H), jnp.float32),
    )(s0, s1, fc_wt, fc_b)


def _tc_combine(cs, att, s0, s1, pred_wt, pred_b):
    def body(cs_ref, att_ref, x0_ref, x1_ref, w_ref, b_ref, out_ref):
        a = att_ref[0, :]
        v0 = jnp.sum(cs_ref[0, :] * a) * (1.0 / N)
        v1 = jnp.sum(cs_ref[1, :] * a) * (1.0 / N)
        m = jnp.maximum(v0, v1)
        e0 = jnp.exp(v0 - m)
        e1 = jnp.exp(v1 - m)
        inv = 1.0 / (e0 + e1)
        b0 = e0 * inv
        b1 = e1 * inv
        z = x0_ref[...] * b0 + x1_ref[...] * b1
        out_ref[...] = jnp.tanh(
            jnp.dot(z, w_ref[...], preferred_element_type=jnp.float32)
            + b_ref[...])

    return pl.pallas_call(
        body,
        grid=(GRID,),
        in_specs=[
            pl.BlockSpec((8, H), lambda i: (0, 0)),
            pl.BlockSpec((1, H), lambda i: (0, 0)),
            pl.BlockSpec((BLK, H), lambda i: (i, 0)),
            pl.BlockSpec((BLK, H), lambda i: (i, 0)),
            pl.BlockSpec((H, H), lambda i: (0, 0)),
            pl.BlockSpec((1, H), lambda i: (0, 0)),
        ],
        out_specs=pl.BlockSpec((BLK, H), lambda i: (i, 0)),
        out_shape=jax.ShapeDtypeStruct((N, H), jnp.float32),
    )(cs, att, s0, s1, pred_wt, pred_b)


def kernel(h0, h1, h2, nei_idx0, nei_idx1, fc_W, fc_b, att, pred_W, pred_b):
    del h0
    idx0 = nei_idx0.astype(jnp.int32).reshape(-1)
    idx1 = nei_idx1.astype(jnp.int32).reshape(-1)
    pad = NPAD * S - idx0.shape[0]
    idx0 = jnp.concatenate([idx0, jnp.zeros((pad,), jnp.int32)])
    idx1 = jnp.concatenate([idx1, jnp.zeros((pad,), jnp.int32)])

    s0, s1 = _sc_gather_sums(h1, h2, idx0, idx1)

    fc_wt = fc_W.T * (1.0 / S)
    pred_wt = pred_W.T * (1.0 / S)
    cs = _tc_colsums(s0, s1, fc_wt, fc_b.reshape(1, H))
    out = _tc_combine(cs, att.reshape(1, H), s0, s1,
                      pred_wt, pred_b.reshape(1, H))
    return out
